# baseline (speedup 1.0000x reference)
import functools
import math

import jax
import jax.numpy as jnp
from jax.experimental import pallas as pl
from jax.experimental.pallas import tpu as pltpu

NEG_INF = -1e30


def _ln_kernel(x_ref, g_ref, b_ref, o_ref, *, eps):
    x = x_ref[...].astype(jnp.float32)
    mu = jnp.mean(x, axis=-1, keepdims=True)
    var = jnp.mean(x * x, axis=-1, keepdims=True) - mu * mu
    y = (x - mu) * jax.lax.rsqrt(var + eps)
    o_ref[...] = (y * g_ref[...] + b_ref[...]).astype(o_ref.dtype)


def _layernorm_bf16(x, g, b, *, eps, block_rows=512):
    rows, hidden = x.shape
    br = math.gcd(rows, block_rows)
    return pl.pallas_call(
        functools.partial(_ln_kernel, eps=eps),
        out_shape=jax.ShapeDtypeStruct((rows, hidden), jnp.bfloat16),
        grid=(rows // br,),
        in_specs=[
            pl.BlockSpec((br, hidden), lambda i: (i, 0)),
            pl.BlockSpec((1, hidden), lambda i: (0, 0)),
            pl.BlockSpec((1, hidden), lambda i: (0, 0)),
        ],
        out_specs=pl.BlockSpec((br, hidden), lambda i: (i, 0)),
        compiler_params=pltpu.CompilerParams(dimension_semantics=("parallel",)),
    )(x, g.reshape(1, hidden), b.reshape(1, hidden))


def _matmul_kernel(x_ref, w_ref, b_ref, o_ref):
    half = x_ref.shape[0] // 2
    w = w_ref[...]
    b = b_ref[...]
    for r in (slice(0, half), slice(half, 2 * half)):
        acc = jnp.dot(x_ref[r, :], w, preferred_element_type=jnp.float32)
        o_ref[r, :] = (acc + b).astype(o_ref.dtype)


def _matmul_bf16(x, w, b, *, block_m=512, block_n=1536):
    m, k = x.shape
    _, n = w.shape
    bm, bn = math.gcd(m, block_m), math.gcd(n, block_n)
    return pl.pallas_call(
        _matmul_kernel,
        out_shape=jax.ShapeDtypeStruct((m, n), jnp.bfloat16),
        grid=(m // bm, n // bn),
        in_specs=[
            pl.BlockSpec((bm, k), lambda i, j: (i, 0)),
            pl.BlockSpec((k, bn), lambda i, j: (0, j)),
            pl.BlockSpec((1, bn), lambda i, j: (0, j)),
        ],
        out_specs=pl.BlockSpec((bm, bn), lambda i, j: (i, j)),
        compiler_params=pltpu.CompilerParams(
            dimension_semantics=("parallel", "parallel")),
    )(x, w, b.reshape(1, n))


def _attn_kernel(slopes_ref, q_ref, k_ref, v_ref, o_ref, *,
                 scale, block_q, chunk, head_dim):
    h = pl.program_id(0)
    qi = pl.program_id(1)
    bq = block_q
    C = chunk
    q = q_ref[...] * jnp.bfloat16(scale)
    slope = slopes_ref[h]
    row_abs = qi * bq + jax.lax.broadcasted_iota(jnp.int32, (bq, 1), 0)

    def stream_init():
        return (jnp.full((bq, 1), NEG_INF, jnp.float32),
                jnp.zeros((bq, 1), jnp.float32),
                jnp.zeros((bq, head_dim), jnp.float32))

    def chunk_update(carry, base):
        m_p, l_p, acc_p = carry
        k = k_ref[pl.ds(base, C), :]
        v = v_ref[pl.ds(base, C), :]
        s = jax.lax.dot_general(q, k, (((1,), (1,)), ((), ())),
                                preferred_element_type=jnp.float32)
        col_abs = base + jax.lax.broadcasted_iota(jnp.int32, (1, C), 1)
        s = s + slope * col_abs.astype(jnp.float32)
        s = jnp.where(col_abs <= row_abs, s, NEG_INF)
        m_n = jnp.maximum(m_p, s.max(axis=-1, keepdims=True))
        alpha = jnp.exp(m_p - m_n)
        p = jnp.exp(s - m_n)
        l_n = alpha * l_p + p.sum(axis=-1, keepdims=True)
        acc_n = alpha * acc_p + jnp.dot(p.astype(v.dtype), v,
                                        preferred_element_type=jnp.float32)
        return (m_n, l_n, acc_n)

    def body(i, carry):
        ca, cb = carry
        base = i * (2 * C)
        return (chunk_update(ca, base), chunk_update(cb, base + C))

    n_iters = (qi * bq) // (2 * C) + 1
    (m_a, l_a, acc_a), (m_b, l_b, acc_b) = jax.lax.fori_loop(
        0, n_iters, body, (stream_init(), stream_init()))
    m = jnp.maximum(m_a, m_b)
    w_a = jnp.exp(m_a - m)
    w_b = jnp.exp(m_b - m)
    l = l_a * w_a + l_b * w_b
    acc = acc_a * w_a + acc_b * w_b
    o_ref[...] = (acc / l).astype(o_ref.dtype)


def _attention(qkv, slopes, *, num_heads, head_dim, scale, block_q=512):
    seq = qkv.shape[0]
    hidden = num_heads * head_dim
    bq = math.gcd(seq, block_q)
    H = num_heads
    return pl.pallas_call(
        functools.partial(_attn_kernel, scale=scale, block_q=bq,
                          chunk=bq, head_dim=head_dim),
        out_shape=jax.ShapeDtypeStruct((seq, hidden), qkv.dtype),
        grid=(num_heads, seq // bq),
        in_specs=[
            pl.BlockSpec(memory_space=pltpu.MemorySpace.SMEM),
            pl.BlockSpec((bq, head_dim), lambda h, qi: (qi, h)),
            pl.BlockSpec((seq, head_dim), lambda h, qi: (0, H + h)),
            pl.BlockSpec((seq, head_dim), lambda h, qi: (0, 2 * H + h)),
        ],
        out_specs=pl.BlockSpec((bq, head_dim), lambda h, qi: (qi, h)),
        compiler_params=pltpu.CompilerParams(
            dimension_semantics=("parallel", "arbitrary")),
    )(slopes, qkv, qkv, qkv)


def _proj_res_ln_kernel(ctx_ref, w_ref, bias_ref, xres_ref, g_ref, b_ref,
                        x2_ref, h2_ref, *, eps):
    acc = jnp.dot(ctx_ref[...], w_ref[...], preferred_element_type=jnp.float32)
    x2 = acc + bias_ref[...] + xres_ref[...]
    x2_ref[...] = x2.astype(x2_ref.dtype)
    mu = jnp.mean(x2, axis=-1, keepdims=True)
    var = jnp.mean(x2 * x2, axis=-1, keepdims=True) - mu * mu
    y = (x2 - mu) * jax.lax.rsqrt(var + eps)
    h2_ref[...] = (y * g_ref[...] + b_ref[...]).astype(h2_ref.dtype)


def _proj_res_ln(ctx, w, bias, xres, g, b, *, eps, block_m=512):
    m, k = ctx.shape
    _, n = w.shape
    bm = math.gcd(m, block_m)
    return pl.pallas_call(
        functools.partial(_proj_res_ln_kernel, eps=eps),
        out_shape=(jax.ShapeDtypeStruct((m, n), jnp.float32),
                   jax.ShapeDtypeStruct((m, n), jnp.bfloat16)),
        grid=(m // bm,),
        in_specs=[
            pl.BlockSpec((bm, k), lambda i: (i, 0)),
            pl.BlockSpec((k, n), lambda i: (0, 0)),
            pl.BlockSpec((1, n), lambda i: (0, 0)),
            pl.BlockSpec((bm, n), lambda i: (i, 0)),
            pl.BlockSpec((1, n), lambda i: (0, 0)),
            pl.BlockSpec((1, n), lambda i: (0, 0)),
        ],
        out_specs=(pl.BlockSpec((bm, n), lambda i: (i, 0)),
                   pl.BlockSpec((bm, n), lambda i: (i, 0))),
        compiler_params=pltpu.CompilerParams(dimension_semantics=("parallel",)),
    )(ctx, w, bias.reshape(1, n), xres, g.reshape(1, n), b.reshape(1, n))


def _mlp_kernel(h2_ref, w1_ref, b1_ref, w2_ref, b2_ref, wp_ref, bp_ref,
                xres_ref, o_ref, acc_ref):
    j = pl.program_id(1)

    @pl.when(j == 0)
    def _():
        acc_ref[...] = jnp.zeros_like(acc_ref)

    half = h2_ref.shape[0] // 2
    w1 = w1_ref[...]
    w2 = w2_ref[...]
    wp = wp_ref[...]
    b1 = b1_ref[...]
    b2 = b2_ref[...]
    for r in (slice(0, half), slice(half, 2 * half)):
        h = h2_ref[r, :]
        t1 = jnp.dot(h, w1, preferred_element_type=jnp.float32) + b1
        t2 = jnp.dot(h, w2, preferred_element_type=jnp.float32) + b2
        g = t1 * (t2 * jax.nn.sigmoid(t2))
        acc_ref[r, :] += jnp.dot(g.astype(jnp.bfloat16), wp,
                                 preferred_element_type=jnp.float32)

    @pl.when(j == pl.num_programs(1) - 1)
    def _():
        o_ref[...] = (acc_ref[...] + bp_ref[...]
                      + xres_ref[...]).astype(o_ref.dtype)


def _mlp(h2, w1, b1, w2, b2, wp, bp, xres, *, block_m=512, block_n=512):
    m, k = h2.shape
    _, inner = w1.shape
    n_out = wp.shape[1]
    bm = math.gcd(m, block_m)
    bn = math.gcd(inner, block_n)
    return pl.pallas_call(
        _mlp_kernel,
        out_shape=jax.ShapeDtypeStruct((m, n_out), jnp.float32),
        grid=(m // bm, inner // bn),
        in_specs=[
            pl.BlockSpec((bm, k), lambda i, j: (i, 0)),
            pl.BlockSpec((k, bn), lambda i, j: (0, j)),
            pl.BlockSpec((1, bn), lambda i, j: (0, j)),
            pl.BlockSpec((k, bn), lambda i, j: (0, j)),
            pl.BlockSpec((1, bn), lambda i, j: (0, j)),
            pl.BlockSpec((bn, n_out), lambda i, j: (j, 0)),
            pl.BlockSpec((1, n_out), lambda i, j: (0, 0)),
            pl.BlockSpec((bm, n_out), lambda i, j: (i, 0)),
        ],
        out_specs=pl.BlockSpec((bm, n_out), lambda i, j: (i, 0)),
        scratch_shapes=[pltpu.VMEM((bm, n_out), jnp.float32)],
        compiler_params=pltpu.CompilerParams(
            dimension_semantics=("parallel", "arbitrary")),
    )(h2, w1, b1.reshape(1, inner), w2, b2.reshape(1, inner),
      wp, bp.reshape(1, n_out), xres)


def kernel(x, ln1_g, ln1_b, c_attn_w, c_attn_b, c_proj_w, c_proj_b,
           ln2_g, ln2_b, c_fc_w, c_fc_b, c_fc2_w, c_fc2_b,
           mlp_proj_w, mlp_proj_b, slopes):
    seq, hidden = x.shape
    num_heads = slopes.shape[0]
    head_dim = hidden // num_heads
    scale = float(head_dim) ** -1.0
    eps = 1e-5
    bf = jnp.bfloat16

    h1 = _layernorm_bf16(x, ln1_g, ln1_b, eps=eps)
    qkv = _matmul_bf16(h1, c_attn_w.astype(bf), c_attn_b)
    ctx = _attention(qkv, slopes, num_heads=num_heads, head_dim=head_dim,
                     scale=scale)
    x2, h2 = _proj_res_ln(ctx, c_proj_w.astype(bf), c_proj_b, x,
                          ln2_g, ln2_b, eps=eps)
    return _mlp(h2, c_fc_w.astype(bf), c_fc_b, c_fc2_w.astype(bf), c_fc2_b,
                mlp_proj_w.astype(bf), mlp_proj_b, x2)


# speedup vs baseline: 1.0327x; 1.0327x over previous
import functools
import math

import jax
import jax.numpy as jnp
from jax.experimental import pallas as pl
from jax.experimental.pallas import tpu as pltpu

NEG_INF = -1e30


def _ln_kernel(x_ref, g_ref, b_ref, o_ref, *, eps):
    x = x_ref[...].astype(jnp.float32)
    mu = jnp.mean(x, axis=-1, keepdims=True)
    var = jnp.mean(x * x, axis=-1, keepdims=True) - mu * mu
    y = (x - mu) * jax.lax.rsqrt(var + eps)
    o_ref[...] = (y * g_ref[...] + b_ref[...]).astype(o_ref.dtype)


def _layernorm_bf16(x, g, b, *, eps, block_rows=512):
    rows, hidden = x.shape
    br = math.gcd(rows, block_rows)
    return pl.pallas_call(
        functools.partial(_ln_kernel, eps=eps),
        out_shape=jax.ShapeDtypeStruct((rows, hidden), jnp.bfloat16),
        grid=(rows // br,),
        in_specs=[
            pl.BlockSpec((br, hidden), lambda i: (i, 0)),
            pl.BlockSpec((1, hidden), lambda i: (0, 0)),
            pl.BlockSpec((1, hidden), lambda i: (0, 0)),
        ],
        out_specs=pl.BlockSpec((br, hidden), lambda i: (i, 0)),
        compiler_params=pltpu.CompilerParams(dimension_semantics=("parallel",)),
    )(x, g.reshape(1, hidden), b.reshape(1, hidden))


def _matmul_kernel(x_ref, w_ref, b_ref, o_ref):
    acc = jnp.dot(x_ref[...], w_ref[...], preferred_element_type=jnp.float32)
    o_ref[...] = (acc + b_ref[...]).astype(o_ref.dtype)


def _matmul_bf16(x, w, b, *, block_m=512, block_n=1536):
    m, k = x.shape
    _, n = w.shape
    bm, bn = math.gcd(m, block_m), math.gcd(n, block_n)
    return pl.pallas_call(
        _matmul_kernel,
        out_shape=jax.ShapeDtypeStruct((m, n), jnp.bfloat16),
        grid=(m // bm, n // bn),
        in_specs=[
            pl.BlockSpec((bm, k), lambda i, j: (i, 0)),
            pl.BlockSpec((k, bn), lambda i, j: (0, j)),
            pl.BlockSpec((1, bn), lambda i, j: (0, j)),
        ],
        out_specs=pl.BlockSpec((bm, bn), lambda i, j: (i, j)),
        compiler_params=pltpu.CompilerParams(
            dimension_semantics=("parallel", "parallel")),
    )(x, w, b.reshape(1, n))


def _attn_kernel(slopes_ref, q_ref, k_ref, v_ref, o_ref, *,
                 scale, block_q, chunk, head_dim):
    h = pl.program_id(0)
    qi = pl.program_id(1)
    bq = block_q
    C = chunk
    q = q_ref[...] * jnp.bfloat16(scale)
    slope = slopes_ref[h]
    row_abs = qi * bq + jax.lax.broadcasted_iota(jnp.int32, (bq, 1), 0)

    def stream_init():
        return (jnp.full((bq, 1), NEG_INF, jnp.float32),
                jnp.zeros((bq, 1), jnp.float32),
                jnp.zeros((bq, head_dim), jnp.float32))

    def chunk_update(carry, base):
        m_p, l_p, acc_p = carry
        k = k_ref[pl.ds(base, C), :]
        v = v_ref[pl.ds(base, C), :]
        s = jax.lax.dot_general(q, k, (((1,), (1,)), ((), ())),
                                preferred_element_type=jnp.float32)
        col_abs = base + jax.lax.broadcasted_iota(jnp.int32, (1, C), 1)
        s = s + slope * col_abs.astype(jnp.float32)
        s = jnp.where(col_abs <= row_abs, s, NEG_INF)
        m_n = jnp.maximum(m_p, s.max(axis=-1, keepdims=True))
        alpha = jnp.exp(m_p - m_n)
        p = jnp.exp(s - m_n)
        l_n = alpha * l_p + p.sum(axis=-1, keepdims=True)
        acc_n = alpha * acc_p + jnp.dot(p.astype(v.dtype), v,
                                        preferred_element_type=jnp.float32)
        return (m_n, l_n, acc_n)

    def body(i, carry):
        ca, cb = carry
        base = i * (2 * C)
        return (chunk_update(ca, base), chunk_update(cb, base + C))

    n_iters = (qi * bq) // (2 * C) + 1
    (m_a, l_a, acc_a), (m_b, l_b, acc_b) = jax.lax.fori_loop(
        0, n_iters, body, (stream_init(), stream_init()))
    m = jnp.maximum(m_a, m_b)
    w_a = jnp.exp(m_a - m)
    w_b = jnp.exp(m_b - m)
    l = l_a * w_a + l_b * w_b
    acc = acc_a * w_a + acc_b * w_b
    o_ref[...] = (acc / l).astype(o_ref.dtype)


def _attention(qkv, slopes, *, num_heads, head_dim, scale, block_q=512):
    seq = qkv.shape[0]
    hidden = num_heads * head_dim
    bq = math.gcd(seq, block_q)
    H = num_heads
    return pl.pallas_call(
        functools.partial(_attn_kernel, scale=scale, block_q=bq,
                          chunk=bq, head_dim=head_dim),
        out_shape=jax.ShapeDtypeStruct((seq, hidden), qkv.dtype),
        grid=(num_heads, seq // bq),
        in_specs=[
            pl.BlockSpec(memory_space=pltpu.MemorySpace.SMEM),
            pl.BlockSpec((bq, head_dim), lambda h, qi: (qi, h)),
            pl.BlockSpec((seq, head_dim), lambda h, qi: (0, H + h)),
            pl.BlockSpec((seq, head_dim), lambda h, qi: (0, 2 * H + h)),
        ],
        out_specs=pl.BlockSpec((bq, head_dim), lambda h, qi: (qi, h)),
        compiler_params=pltpu.CompilerParams(
            dimension_semantics=("parallel", "arbitrary")),
    )(slopes, qkv, qkv, qkv)


def _block2_kernel(ctx_ref, wproj_ref, bproj_ref, x_ref, g2_ref, bn2_ref,
                   w1_ref, b1_ref, w2_ref, b2_ref, wp_ref, bp_ref,
                   o_ref, h2_sc, x2_sc, acc_ref, *, eps):
    j = pl.program_id(1)

    @pl.when(j == 0)
    def _():
        pr = jnp.dot(ctx_ref[...], wproj_ref[...],
                     preferred_element_type=jnp.float32)
        x2 = pr + bproj_ref[...] + x_ref[...]
        x2_sc[...] = x2
        mu = jnp.mean(x2, axis=-1, keepdims=True)
        var = jnp.mean(x2 * x2, axis=-1, keepdims=True) - mu * mu
        y = (x2 - mu) * jax.lax.rsqrt(var + eps)
        h2_sc[...] = (y * g2_ref[...] + bn2_ref[...]).astype(h2_sc.dtype)
        acc_ref[...] = jnp.zeros_like(acc_ref)

    h = h2_sc[...]
    t1 = jnp.dot(h, w1_ref[...], preferred_element_type=jnp.float32) + b1_ref[...]
    t2 = jnp.dot(h, w2_ref[...], preferred_element_type=jnp.float32) + b2_ref[...]
    g = t1 * (t2 * jax.nn.sigmoid(t2))
    acc_ref[...] += jnp.dot(g.astype(jnp.bfloat16), wp_ref[...],
                            preferred_element_type=jnp.float32)

    @pl.when(j == pl.num_programs(1) - 1)
    def _():
        o_ref[...] = (acc_ref[...] + bp_ref[...]
                      + x2_sc[...]).astype(o_ref.dtype)


def _block2(ctx, wproj, bproj, x, g2, bn2, w1, b1, w2, b2, wp, bp, *,
            eps, block_m=512, block_n=512):
    m, k = ctx.shape
    _, inner = w1.shape
    n_out = wp.shape[1]
    bm = math.gcd(m, block_m)
    bn = math.gcd(inner, block_n)
    return pl.pallas_call(
        functools.partial(_block2_kernel, eps=eps),
        out_shape=jax.ShapeDtypeStruct((m, n_out), jnp.float32),
        grid=(m // bm, inner // bn),
        in_specs=[
            pl.BlockSpec((bm, k), lambda i, j: (i, 0)),
            pl.BlockSpec((k, k), lambda i, j: (0, 0)),
            pl.BlockSpec((1, k), lambda i, j: (0, 0)),
            pl.BlockSpec((bm, k), lambda i, j: (i, 0)),
            pl.BlockSpec((1, k), lambda i, j: (0, 0)),
            pl.BlockSpec((1, k), lambda i, j: (0, 0)),
            pl.BlockSpec((k, bn), lambda i, j: (0, j)),
            pl.BlockSpec((1, bn), lambda i, j: (0, j)),
            pl.BlockSpec((k, bn), lambda i, j: (0, j)),
            pl.BlockSpec((1, bn), lambda i, j: (0, j)),
            pl.BlockSpec((bn, n_out), lambda i, j: (j, 0)),
            pl.BlockSpec((1, n_out), lambda i, j: (0, 0)),
        ],
        out_specs=pl.BlockSpec((bm, n_out), lambda i, j: (i, 0)),
        scratch_shapes=[pltpu.VMEM((bm, k), jnp.bfloat16),
                        pltpu.VMEM((bm, k), jnp.float32),
                        pltpu.VMEM((bm, n_out), jnp.float32)],
        compiler_params=pltpu.CompilerParams(
            dimension_semantics=("parallel", "arbitrary")),
    )(ctx, wproj, bproj.reshape(1, k), x, g2.reshape(1, k), bn2.reshape(1, k),
      w1, b1.reshape(1, inner), w2, b2.reshape(1, inner),
      wp, bp.reshape(1, n_out))


def kernel(x, ln1_g, ln1_b, c_attn_w, c_attn_b, c_proj_w, c_proj_b,
           ln2_g, ln2_b, c_fc_w, c_fc_b, c_fc2_w, c_fc2_b,
           mlp_proj_w, mlp_proj_b, slopes):
    seq, hidden = x.shape
    num_heads = slopes.shape[0]
    head_dim = hidden // num_heads
    scale = float(head_dim) ** -1.0
    eps = 1e-5
    bf = jnp.bfloat16

    h1 = _layernorm_bf16(x, ln1_g, ln1_b, eps=eps)
    qkv = _matmul_bf16(h1, c_attn_w.astype(bf), c_attn_b)
    ctx = _attention(qkv, slopes, num_heads=num_heads, head_dim=head_dim,
                     scale=scale)
    return _block2(ctx, c_proj_w.astype(bf), c_proj_b, x, ln2_g, ln2_b,
                   c_fc_w.astype(bf), c_fc_b, c_fc2_w.astype(bf), c_fc2_b,
                   mlp_proj_w.astype(bf), mlp_proj_b, eps=eps)


# speedup vs baseline: 1.0391x; 1.0061x over previous
import functools
import math

import jax
import jax.numpy as jnp
from jax.experimental import pallas as pl
from jax.experimental.pallas import tpu as pltpu

NEG_INF = -1e30


def _ln_kernel(x_ref, g_ref, b_ref, o_ref, *, eps):
    x = x_ref[...].astype(jnp.float32)
    mu = jnp.mean(x, axis=-1, keepdims=True)
    var = jnp.mean(x * x, axis=-1, keepdims=True) - mu * mu
    y = (x - mu) * jax.lax.rsqrt(var + eps)
    o_ref[...] = (y * g_ref[...] + b_ref[...]).astype(o_ref.dtype)


def _layernorm_bf16(x, g, b, *, eps, block_rows=512):
    rows, hidden = x.shape
    br = math.gcd(rows, block_rows)
    return pl.pallas_call(
        functools.partial(_ln_kernel, eps=eps),
        out_shape=jax.ShapeDtypeStruct((rows, hidden), jnp.bfloat16),
        grid=(rows // br,),
        in_specs=[
            pl.BlockSpec((br, hidden), lambda i: (i, 0)),
            pl.BlockSpec((1, hidden), lambda i: (0, 0)),
            pl.BlockSpec((1, hidden), lambda i: (0, 0)),
        ],
        out_specs=pl.BlockSpec((br, hidden), lambda i: (i, 0)),
        compiler_params=pltpu.CompilerParams(dimension_semantics=("parallel",)),
    )(x, g.reshape(1, hidden), b.reshape(1, hidden))


def _ln_matmul_kernel(x_ref, g_ref, b_ref, w_ref, bias_ref, o_ref, *, eps):
    x = x_ref[...]
    mu = jnp.mean(x, axis=-1, keepdims=True)
    var = jnp.mean(x * x, axis=-1, keepdims=True) - mu * mu
    y = (x - mu) * jax.lax.rsqrt(var + eps)
    y = (y * g_ref[...] + b_ref[...]).astype(jnp.bfloat16)
    acc = jnp.dot(y, w_ref[...], preferred_element_type=jnp.float32)
    o_ref[...] = (acc + bias_ref[...]).astype(o_ref.dtype)


def _ln_matmul_bf16(x, g, b, w, bias, *, eps, block_m=512, block_n=1536):
    m, k = x.shape
    _, n = w.shape
    bm, bn = math.gcd(m, block_m), math.gcd(n, block_n)
    return pl.pallas_call(
        functools.partial(_ln_matmul_kernel, eps=eps),
        out_shape=jax.ShapeDtypeStruct((m, n), jnp.bfloat16),
        grid=(m // bm, n // bn),
        in_specs=[
            pl.BlockSpec((bm, k), lambda i, j: (i, 0)),
            pl.BlockSpec((1, k), lambda i, j: (0, 0)),
            pl.BlockSpec((1, k), lambda i, j: (0, 0)),
            pl.BlockSpec((k, bn), lambda i, j: (0, j)),
            pl.BlockSpec((1, bn), lambda i, j: (0, j)),
        ],
        out_specs=pl.BlockSpec((bm, bn), lambda i, j: (i, j)),
        compiler_params=pltpu.CompilerParams(
            dimension_semantics=("parallel", "parallel")),
    )(x, g.reshape(1, k), b.reshape(1, k), w, bias.reshape(1, n))


def _attn_kernel(slopes_ref, q_ref, k_ref, v_ref, o_ref, *,
                 scale, block_q, chunk, head_dim):
    h = pl.program_id(0)
    qi = pl.program_id(1)
    bq = block_q
    C = chunk
    q = q_ref[...] * jnp.bfloat16(scale)
    slope = slopes_ref[h]
    row_abs = qi * bq + jax.lax.broadcasted_iota(jnp.int32, (bq, 1), 0)

    def stream_init():
        return (jnp.full((bq, 1), NEG_INF, jnp.float32),
                jnp.zeros((bq, 1), jnp.float32),
                jnp.zeros((bq, head_dim), jnp.float32))

    def chunk_update(carry, base):
        m_p, l_p, acc_p = carry
        k = k_ref[pl.ds(base, C), :]
        v = v_ref[pl.ds(base, C), :]
        s = jax.lax.dot_general(q, k, (((1,), (1,)), ((), ())),
                                preferred_element_type=jnp.float32)
        col_abs = base + jax.lax.broadcasted_iota(jnp.int32, (1, C), 1)
        s = s + slope * col_abs.astype(jnp.float32)
        s = jnp.where(col_abs <= row_abs, s, NEG_INF)
        m_n = jnp.maximum(m_p, s.max(axis=-1, keepdims=True))
        alpha = jnp.exp(m_p - m_n)
        p = jnp.exp(s - m_n)
        l_n = alpha * l_p + p.sum(axis=-1, keepdims=True)
        acc_n = alpha * acc_p + jnp.dot(p.astype(v.dtype), v,
                                        preferred_element_type=jnp.float32)
        return (m_n, l_n, acc_n)

    def body(i, carry):
        ca, cb = carry
        base = i * (2 * C)
        return (chunk_update(ca, base), chunk_update(cb, base + C))

    n_iters = (qi * bq) // (2 * C) + 1
    (m_a, l_a, acc_a), (m_b, l_b, acc_b) = jax.lax.fori_loop(
        0, n_iters, body, (stream_init(), stream_init()))
    m = jnp.maximum(m_a, m_b)
    w_a = jnp.exp(m_a - m)
    w_b = jnp.exp(m_b - m)
    l = l_a * w_a + l_b * w_b
    acc = acc_a * w_a + acc_b * w_b
    o_ref[...] = (acc / l).astype(o_ref.dtype)


def _attention(qkv, slopes, *, num_heads, head_dim, scale, block_q=512):
    seq = qkv.shape[0]
    hidden = num_heads * head_dim
    bq = math.gcd(seq, block_q)
    H = num_heads
    return pl.pallas_call(
        functools.partial(_attn_kernel, scale=scale, block_q=bq,
                          chunk=bq, head_dim=head_dim),
        out_shape=jax.ShapeDtypeStruct((seq, hidden), qkv.dtype),
        grid=(num_heads, seq // bq),
        in_specs=[
            pl.BlockSpec(memory_space=pltpu.MemorySpace.SMEM),
            pl.BlockSpec((bq, head_dim), lambda h, qi: (qi, h)),
            pl.BlockSpec((seq, head_dim), lambda h, qi: (0, H + h)),
            pl.BlockSpec((seq, head_dim), lambda h, qi: (0, 2 * H + h)),
        ],
        out_specs=pl.BlockSpec((bq, head_dim), lambda h, qi: (qi, h)),
        compiler_params=pltpu.CompilerParams(
            dimension_semantics=("parallel", "arbitrary")),
    )(slopes, qkv, qkv, qkv)


def _block2_kernel(ctx_ref, wproj_ref, bproj_ref, x_ref, g2_ref, bn2_ref,
                   w1_ref, b1_ref, w2_ref, b2_ref, wp_ref, bp_ref,
                   o_ref, h2_sc, x2_sc, acc_ref, *, eps):
    j = pl.program_id(1)

    @pl.when(j == 0)
    def _():
        pr = jnp.dot(ctx_ref[...], wproj_ref[...],
                     preferred_element_type=jnp.float32)
        x2 = pr + bproj_ref[...] + x_ref[...]
        x2_sc[...] = x2
        mu = jnp.mean(x2, axis=-1, keepdims=True)
        var = jnp.mean(x2 * x2, axis=-1, keepdims=True) - mu * mu
        y = (x2 - mu) * jax.lax.rsqrt(var + eps)
        h2_sc[...] = (y * g2_ref[...] + bn2_ref[...]).astype(h2_sc.dtype)
        acc_ref[...] = jnp.zeros_like(acc_ref)

    h = h2_sc[...]
    t1 = jnp.dot(h, w1_ref[...], preferred_element_type=jnp.float32) + b1_ref[...]
    t2 = jnp.dot(h, w2_ref[...], preferred_element_type=jnp.float32) + b2_ref[...]
    g = t1 * (t2 * jax.nn.sigmoid(t2))
    acc_ref[...] += jnp.dot(g.astype(jnp.bfloat16), wp_ref[...],
                            preferred_element_type=jnp.float32)

    @pl.when(j == pl.num_programs(1) - 1)
    def _():
        o_ref[...] = (acc_ref[...] + bp_ref[...]
                      + x2_sc[...]).astype(o_ref.dtype)


def _block2(ctx, wproj, bproj, x, g2, bn2, w1, b1, w2, b2, wp, bp, *,
            eps, block_m=512, block_n=512):
    m, k = ctx.shape
    _, inner = w1.shape
    n_out = wp.shape[1]
    bm = math.gcd(m, block_m)
    bn = math.gcd(inner, block_n)
    return pl.pallas_call(
        functools.partial(_block2_kernel, eps=eps),
        out_shape=jax.ShapeDtypeStruct((m, n_out), jnp.float32),
        grid=(m // bm, inner // bn),
        in_specs=[
            pl.BlockSpec((bm, k), lambda i, j: (i, 0)),
            pl.BlockSpec((k, k), lambda i, j: (0, 0)),
            pl.BlockSpec((1, k), lambda i, j: (0, 0)),
            pl.BlockSpec((bm, k), lambda i, j: (i, 0)),
            pl.BlockSpec((1, k), lambda i, j: (0, 0)),
            pl.BlockSpec((1, k), lambda i, j: (0, 0)),
            pl.BlockSpec((k, bn), lambda i, j: (0, j)),
            pl.BlockSpec((1, bn), lambda i, j: (0, j)),
            pl.BlockSpec((k, bn), lambda i, j: (0, j)),
            pl.BlockSpec((1, bn), lambda i, j: (0, j)),
            pl.BlockSpec((bn, n_out), lambda i, j: (j, 0)),
            pl.BlockSpec((1, n_out), lambda i, j: (0, 0)),
        ],
        out_specs=pl.BlockSpec((bm, n_out), lambda i, j: (i, 0)),
        scratch_shapes=[pltpu.VMEM((bm, k), jnp.bfloat16),
                        pltpu.VMEM((bm, k), jnp.float32),
                        pltpu.VMEM((bm, n_out), jnp.float32)],
        compiler_params=pltpu.CompilerParams(
            dimension_semantics=("parallel", "arbitrary")),
    )(ctx, wproj, bproj.reshape(1, k), x, g2.reshape(1, k), bn2.reshape(1, k),
      w1, b1.reshape(1, inner), w2, b2.reshape(1, inner),
      wp, bp.reshape(1, n_out))


def kernel(x, ln1_g, ln1_b, c_attn_w, c_attn_b, c_proj_w, c_proj_b,
           ln2_g, ln2_b, c_fc_w, c_fc_b, c_fc2_w, c_fc2_b,
           mlp_proj_w, mlp_proj_b, slopes):
    seq, hidden = x.shape
    num_heads = slopes.shape[0]
    head_dim = hidden // num_heads
    scale = float(head_dim) ** -1.0
    eps = 1e-5
    bf = jnp.bfloat16

    qkv = _ln_matmul_bf16(x, ln1_g, ln1_b, c_attn_w.astype(bf), c_attn_b,
                          eps=eps)
    ctx = _attention(qkv, slopes, num_heads=num_heads, head_dim=head_dim,
                     scale=scale)
    return _block2(ctx, c_proj_w.astype(bf), c_proj_b, x, ln2_g, ln2_b,
                   c_fc_w.astype(bf), c_fc_b, c_fc2_w.astype(bf), c_fc2_b,
                   mlp_proj_w.astype(bf), mlp_proj_b, eps=eps)


# speedup vs baseline: 1.1271x; 1.0847x over previous
import functools
import math

import jax
import jax.numpy as jnp
from jax.experimental import pallas as pl
from jax.experimental.pallas import tpu as pltpu

NEG_INF = -1e30


def _ln_matmul_kernel(x_ref, g_ref, b_ref, w_ref, bias_ref, o_ref, *, eps):
    x = x_ref[...]
    mu = jnp.mean(x, axis=-1, keepdims=True)
    var = jnp.mean(x * x, axis=-1, keepdims=True) - mu * mu
    y = (x - mu) * jax.lax.rsqrt(var + eps)
    y = (y * g_ref[...] + b_ref[...]).astype(jnp.bfloat16)
    acc = jnp.dot(y, w_ref[...], preferred_element_type=jnp.float32)
    o_ref[...] = (acc + bias_ref[...]).astype(o_ref.dtype)


def _ln_matmul_bf16(x, g, b, w, bias, *, eps, block_m=512, block_n=1536):
    m, k = x.shape
    _, n = w.shape
    bm, bn = math.gcd(m, block_m), math.gcd(n, block_n)
    return pl.pallas_call(
        functools.partial(_ln_matmul_kernel, eps=eps),
        out_shape=jax.ShapeDtypeStruct((m, n), jnp.bfloat16),
        grid=(m // bm, n // bn),
        in_specs=[
            pl.BlockSpec((bm, k), lambda i, j: (i, 0)),
            pl.BlockSpec((1, k), lambda i, j: (0, 0)),
            pl.BlockSpec((1, k), lambda i, j: (0, 0)),
            pl.BlockSpec((k, bn), lambda i, j: (0, j)),
            pl.BlockSpec((1, bn), lambda i, j: (0, j)),
        ],
        out_specs=pl.BlockSpec((bm, bn), lambda i, j: (i, j)),
        compiler_params=pltpu.CompilerParams(
            dimension_semantics=("parallel", "parallel")),
    )(x, g.reshape(1, k), b.reshape(1, k), w, bias.reshape(1, n))


def _attn_kernel(slopes_ref, q_ref, k_ref, v_ref, o_ref, *,
                 scale, block_q, chunk, head_dim):
    h = pl.program_id(0)
    qi = pl.program_id(1)
    bq = block_q
    C = chunk
    q = q_ref[...] * jnp.bfloat16(scale)
    slope = slopes_ref[h]
    row_abs = qi * bq + jax.lax.broadcasted_iota(jnp.int32, (bq, 1), 0)

    def stream_init():
        return (jnp.full((bq, 1), NEG_INF, jnp.float32),
                jnp.zeros((bq, 1), jnp.float32),
                jnp.zeros((bq, head_dim), jnp.float32))

    def chunk_update(carry, base, masked):
        m_p, l_p, acc_p = carry
        k = k_ref[pl.ds(base, C), :]
        v = v_ref[pl.ds(base, C), :]
        s = jax.lax.dot_general(q, k, (((1,), (1,)), ((), ())),
                                preferred_element_type=jnp.float32)
        col_abs = base + jax.lax.broadcasted_iota(jnp.int32, (1, C), 1)
        s = s + slope * col_abs.astype(jnp.float32)
        if masked:
            s = jnp.where(col_abs <= row_abs, s, NEG_INF)
        m_n = jnp.maximum(m_p, s.max(axis=-1, keepdims=True))
        alpha = jnp.exp(m_p - m_n)
        p = jnp.exp(s - m_n)
        l_n = alpha * l_p + p.sum(axis=-1, keepdims=True)
        acc_n = alpha * acc_p + jnp.dot(p.astype(v.dtype), v,
                                        preferred_element_type=jnp.float32)
        return (m_n, l_n, acc_n)

    def body(i, carry):
        ca, cb = carry
        base = i * (2 * C)
        return (chunk_update(ca, base, False), chunk_update(cb, base + C, False))

    n_full = (qi * bq) // (2 * C)
    c_a, c_b = jax.lax.fori_loop(0, n_full, body,
                                 (stream_init(), stream_init()))
    diag = n_full * (2 * C)
    m_a, l_a, acc_a = chunk_update(c_a, diag, True)
    m_b, l_b, acc_b = chunk_update(c_b, diag + C, True)
    m = jnp.maximum(m_a, m_b)
    w_a = jnp.exp(m_a - m)
    w_b = jnp.exp(m_b - m)
    l = l_a * w_a + l_b * w_b
    acc = acc_a * w_a + acc_b * w_b
    o_ref[...] = (acc / l).astype(o_ref.dtype)


def _attention(qkv, slopes, *, num_heads, head_dim, scale, block_q=1024):
    seq = qkv.shape[0]
    hidden = num_heads * head_dim
    bq = math.gcd(seq, block_q)
    H = num_heads
    return pl.pallas_call(
        functools.partial(_attn_kernel, scale=scale, block_q=bq,
                          chunk=min(512, bq), head_dim=head_dim),
        out_shape=jax.ShapeDtypeStruct((seq, hidden), qkv.dtype),
        grid=(num_heads, seq // bq),
        in_specs=[
            pl.BlockSpec(memory_space=pltpu.MemorySpace.SMEM),
            pl.BlockSpec((bq, head_dim), lambda h, qi: (qi, h)),
            pl.BlockSpec((seq, head_dim), lambda h, qi: (0, H + h)),
            pl.BlockSpec((seq, head_dim), lambda h, qi: (0, 2 * H + h)),
        ],
        out_specs=pl.BlockSpec((bq, head_dim), lambda h, qi: (qi, h)),
        compiler_params=pltpu.CompilerParams(
            dimension_semantics=("parallel", "arbitrary")),
    )(slopes, qkv, qkv, qkv)


def _block2_kernel(ctx_ref, wproj_ref, bproj_ref, x_ref, g2_ref, bn2_ref,
                   w1_ref, b1_ref, w2_ref, b2_ref, wp_ref, bp_ref,
                   o_ref, h2_sc, x2_sc, acc_ref, *, eps):
    j = pl.program_id(1)

    @pl.when(j == 0)
    def _():
        pr = jnp.dot(ctx_ref[...], wproj_ref[...],
                     preferred_element_type=jnp.float32)
        x2 = pr + bproj_ref[...] + x_ref[...]
        x2_sc[...] = x2
        mu = jnp.mean(x2, axis=-1, keepdims=True)
        var = jnp.mean(x2 * x2, axis=-1, keepdims=True) - mu * mu
        y = (x2 - mu) * jax.lax.rsqrt(var + eps)
        h2_sc[...] = (y * g2_ref[...] + bn2_ref[...]).astype(h2_sc.dtype)
        acc_ref[...] = jnp.zeros_like(acc_ref)

    h = h2_sc[...]
    t1 = jnp.dot(h, w1_ref[...], preferred_element_type=jnp.float32) + b1_ref[...]
    t2 = jnp.dot(h, w2_ref[...], preferred_element_type=jnp.float32) + b2_ref[...]
    g = t1 * (t2 * jax.nn.sigmoid(t2))
    acc_ref[...] += jnp.dot(g.astype(jnp.bfloat16), wp_ref[...],
                            preferred_element_type=jnp.float32)

    @pl.when(j == pl.num_programs(1) - 1)
    def _():
        o_ref[...] = (acc_ref[...] + bp_ref[...]
                      + x2_sc[...]).astype(o_ref.dtype)


def _block2(ctx, wproj, bproj, x, g2, bn2, w1, b1, w2, b2, wp, bp, *,
            eps, block_m=512, block_n=512):
    m, k = ctx.shape
    _, inner = w1.shape
    n_out = wp.shape[1]
    bm = math.gcd(m, block_m)
    bn = math.gcd(inner, block_n)
    return pl.pallas_call(
        functools.partial(_block2_kernel, eps=eps),
        out_shape=jax.ShapeDtypeStruct((m, n_out), jnp.float32),
        grid=(m // bm, inner // bn),
        in_specs=[
            pl.BlockSpec((bm, k), lambda i, j: (i, 0)),
            pl.BlockSpec((k, k), lambda i, j: (0, 0)),
            pl.BlockSpec((1, k), lambda i, j: (0, 0)),
            pl.BlockSpec((bm, k), lambda i, j: (i, 0)),
            pl.BlockSpec((1, k), lambda i, j: (0, 0)),
            pl.BlockSpec((1, k), lambda i, j: (0, 0)),
            pl.BlockSpec((k, bn), lambda i, j: (0, j)),
            pl.BlockSpec((1, bn), lambda i, j: (0, j)),
            pl.BlockSpec((k, bn), lambda i, j: (0, j)),
            pl.BlockSpec((1, bn), lambda i, j: (0, j)),
            pl.BlockSpec((bn, n_out), lambda i, j: (j, 0)),
            pl.BlockSpec((1, n_out), lambda i, j: (0, 0)),
        ],
        out_specs=pl.BlockSpec((bm, n_out), lambda i, j: (i, 0)),
        scratch_shapes=[pltpu.VMEM((bm, k), jnp.bfloat16),
                        pltpu.VMEM((bm, k), jnp.float32),
                        pltpu.VMEM((bm, n_out), jnp.float32)],
        compiler_params=pltpu.CompilerParams(
            dimension_semantics=("parallel", "arbitrary")),
    )(ctx, wproj, bproj.reshape(1, k), x, g2.reshape(1, k), bn2.reshape(1, k),
      w1, b1.reshape(1, inner), w2, b2.reshape(1, inner),
      wp, bp.reshape(1, n_out))


def kernel(x, ln1_g, ln1_b, c_attn_w, c_attn_b, c_proj_w, c_proj_b,
           ln2_g, ln2_b, c_fc_w, c_fc_b, c_fc2_w, c_fc2_b,
           mlp_proj_w, mlp_proj_b, slopes):
    seq, hidden = x.shape
    num_heads = slopes.shape[0]
    head_dim = hidden // num_heads
    scale = float(head_dim) ** -1.0
    eps = 1e-5
    bf = jnp.bfloat16

    qkv = _ln_matmul_bf16(x, ln1_g, ln1_b, c_attn_w.astype(bf), c_attn_b,
                          eps=eps)
    ctx = _attention(qkv, slopes, num_heads=num_heads, head_dim=head_dim,
                     scale=scale)
    return _block2(ctx, c_proj_w.astype(bf), c_proj_b, x, ln2_g, ln2_b,
                   c_fc_w.astype(bf), c_fc_b, c_fc2_w.astype(bf), c_fc2_b,
                   mlp_proj_w.astype(bf), mlp_proj_b, eps=eps)


# speedup vs baseline: 1.1348x; 1.0069x over previous
import functools
import math

import jax
import jax.numpy as jnp
from jax.experimental import pallas as pl
from jax.experimental.pallas import tpu as pltpu

NEG_INF = -1e30


def _ln_matmul_kernel(x_ref, g_ref, b_ref, w_ref, bias_ref, o_ref, *, eps):
    x = x_ref[...]
    mu = jnp.mean(x, axis=-1, keepdims=True)
    var = jnp.mean(x * x, axis=-1, keepdims=True) - mu * mu
    y = (x - mu) * jax.lax.rsqrt(var + eps)
    y = (y * g_ref[...] + b_ref[...]).astype(jnp.bfloat16)
    acc = jnp.dot(y, w_ref[...], preferred_element_type=jnp.float32)
    o_ref[...] = (acc + bias_ref[...]).astype(o_ref.dtype)


def _ln_matmul_bf16(x, g, b, w, bias, *, eps, block_m=512, block_n=2048):
    m, k = x.shape
    _, n = w.shape
    bm, bn = math.gcd(m, block_m), math.gcd(n, block_n)
    return pl.pallas_call(
        functools.partial(_ln_matmul_kernel, eps=eps),
        out_shape=jax.ShapeDtypeStruct((m, n), jnp.bfloat16),
        grid=(m // bm, n // bn),
        in_specs=[
            pl.BlockSpec((bm, k), lambda i, j: (i, 0)),
            pl.BlockSpec((1, k), lambda i, j: (0, 0)),
            pl.BlockSpec((1, k), lambda i, j: (0, 0)),
            pl.BlockSpec((k, bn), lambda i, j: (0, j)),
            pl.BlockSpec((1, bn), lambda i, j: (0, j)),
        ],
        out_specs=pl.BlockSpec((bm, bn), lambda i, j: (i, j)),
        compiler_params=pltpu.CompilerParams(
            dimension_semantics=("parallel", "parallel")),
    )(x, g.reshape(1, k), b.reshape(1, k), w, bias.reshape(1, n))


def _attn_kernel(slopes_ref, q_ref, k_ref, v_ref, o_ref, *,
                 scale, block_q, chunk, head_dim):
    h = pl.program_id(0)
    qi = pl.program_id(1)
    bq = block_q
    C = chunk
    q = q_ref[...] * jnp.bfloat16(scale)
    slope = slopes_ref[h]
    row_abs = qi * bq + jax.lax.broadcasted_iota(jnp.int32, (bq, 1), 0)

    def stream_init():
        return (jnp.full((bq, 1), NEG_INF, jnp.float32),
                jnp.zeros((bq, 1), jnp.float32),
                jnp.zeros((bq, head_dim), jnp.float32))

    def chunk_update(carry, base, masked):
        m_p, l_p, acc_p = carry
        k = k_ref[pl.ds(base, C), :]
        v = v_ref[pl.ds(base, C), :]
        s = jax.lax.dot_general(q, k, (((1,), (1,)), ((), ())),
                                preferred_element_type=jnp.float32)
        col_abs = base + jax.lax.broadcasted_iota(jnp.int32, (1, C), 1)
        s = s + slope * col_abs.astype(jnp.float32)
        if masked:
            s = jnp.where(col_abs <= row_abs, s, NEG_INF)
        m_n = jnp.maximum(m_p, s.max(axis=-1, keepdims=True))
        alpha = jnp.exp(m_p - m_n)
        p = jnp.exp(s - m_n)
        l_n = alpha * l_p + p.sum(axis=-1, keepdims=True)
        acc_n = alpha * acc_p + jnp.dot(p.astype(v.dtype), v,
                                        preferred_element_type=jnp.float32)
        return (m_n, l_n, acc_n)

    def body(i, carry):
        ca, cb = carry
        base = i * (2 * C)
        return (chunk_update(ca, base, False), chunk_update(cb, base + C, False))

    n_full = (qi * bq) // (2 * C)
    c_a, c_b = jax.lax.fori_loop(0, n_full, body,
                                 (stream_init(), stream_init()))
    diag = n_full * (2 * C)
    m_a, l_a, acc_a = chunk_update(c_a, diag, True)
    m_b, l_b, acc_b = chunk_update(c_b, diag + C, True)
    m = jnp.maximum(m_a, m_b)
    w_a = jnp.exp(m_a - m)
    w_b = jnp.exp(m_b - m)
    l = l_a * w_a + l_b * w_b
    acc = acc_a * w_a + acc_b * w_b
    o_ref[...] = (acc / l).astype(o_ref.dtype)


def _attention(qkv, slopes, *, num_heads, head_dim, scale, block_q=1024):
    seq = qkv.shape[0]
    hidden = num_heads * head_dim
    bq = math.gcd(seq, block_q)
    H = num_heads
    return pl.pallas_call(
        functools.partial(_attn_kernel, scale=scale, block_q=bq,
                          chunk=min(512, bq), head_dim=head_dim),
        out_shape=jax.ShapeDtypeStruct((seq, hidden), qkv.dtype),
        grid=(num_heads, seq // bq),
        in_specs=[
            pl.BlockSpec(memory_space=pltpu.MemorySpace.SMEM),
            pl.BlockSpec((bq, head_dim), lambda h, qi: (qi, h)),
            pl.BlockSpec((seq, head_dim), lambda h, qi: (0, H + h)),
            pl.BlockSpec((seq, head_dim), lambda h, qi: (0, 2 * H + h)),
        ],
        out_specs=pl.BlockSpec((bq, head_dim), lambda h, qi: (qi, h)),
        compiler_params=pltpu.CompilerParams(
            dimension_semantics=("parallel", "arbitrary")),
    )(slopes, qkv, qkv, qkv)


def _block2_kernel(ctx_ref, wproj_ref, bproj_ref, x_ref, g2_ref, bn2_ref,
                   w1_ref, b1_ref, w2_ref, b2_ref, wp_ref, bp_ref,
                   o_ref, h2_sc, x2_sc, acc_ref, *, eps):
    j = pl.program_id(1)

    @pl.when(j == 0)
    def _():
        pr = jnp.dot(ctx_ref[...], wproj_ref[...],
                     preferred_element_type=jnp.float32)
        x2 = pr + bproj_ref[...] + x_ref[...]
        x2_sc[...] = x2
        mu = jnp.mean(x2, axis=-1, keepdims=True)
        var = jnp.mean(x2 * x2, axis=-1, keepdims=True) - mu * mu
        y = (x2 - mu) * jax.lax.rsqrt(var + eps)
        h2_sc[...] = (y * g2_ref[...] + bn2_ref[...]).astype(h2_sc.dtype)
        acc_ref[...] = jnp.zeros_like(acc_ref)

    h = h2_sc[...]
    t1 = jnp.dot(h, w1_ref[...], preferred_element_type=jnp.float32) + b1_ref[...]
    t2 = jnp.dot(h, w2_ref[...], preferred_element_type=jnp.float32) + b2_ref[...]
    g = t1 * (t2 * jax.nn.sigmoid(t2))
    acc_ref[...] += jnp.dot(g.astype(jnp.bfloat16), wp_ref[...],
                            preferred_element_type=jnp.float32)

    @pl.when(j == pl.num_programs(1) - 1)
    def _():
        o_ref[...] = (acc_ref[...] + bp_ref[...]
                      + x2_sc[...]).astype(o_ref.dtype)


def _block2(ctx, wproj, bproj, x, g2, bn2, w1, b1, w2, b2, wp, bp, *,
            eps, block_m=512, block_n=512):
    m, k = ctx.shape
    _, inner = w1.shape
    n_out = wp.shape[1]
    bm = math.gcd(m, block_m)
    bn = math.gcd(inner, block_n)
    return pl.pallas_call(
        functools.partial(_block2_kernel, eps=eps),
        out_shape=jax.ShapeDtypeStruct((m, n_out), jnp.float32),
        grid=(m // bm, inner // bn),
        in_specs=[
            pl.BlockSpec((bm, k), lambda i, j: (i, 0)),
            pl.BlockSpec((k, k), lambda i, j: (0, 0)),
            pl.BlockSpec((1, k), lambda i, j: (0, 0)),
            pl.BlockSpec((bm, k), lambda i, j: (i, 0)),
            pl.BlockSpec((1, k), lambda i, j: (0, 0)),
            pl.BlockSpec((1, k), lambda i, j: (0, 0)),
            pl.BlockSpec((k, bn), lambda i, j: (0, j)),
            pl.BlockSpec((1, bn), lambda i, j: (0, j)),
            pl.BlockSpec((k, bn), lambda i, j: (0, j)),
            pl.BlockSpec((1, bn), lambda i, j: (0, j)),
            pl.BlockSpec((bn, n_out), lambda i, j: (j, 0)),
            pl.BlockSpec((1, n_out), lambda i, j: (0, 0)),
        ],
        out_specs=pl.BlockSpec((bm, n_out), lambda i, j: (i, 0)),
        scratch_shapes=[pltpu.VMEM((bm, k), jnp.bfloat16),
                        pltpu.VMEM((bm, k), jnp.float32),
                        pltpu.VMEM((bm, n_out), jnp.float32)],
        compiler_params=pltpu.CompilerParams(
            dimension_semantics=("parallel", "arbitrary")),
    )(ctx, wproj, bproj.reshape(1, k), x, g2.reshape(1, k), bn2.reshape(1, k),
      w1, b1.reshape(1, inner), w2, b2.reshape(1, inner),
      wp, bp.reshape(1, n_out))


def kernel(x, ln1_g, ln1_b, c_attn_w, c_attn_b, c_proj_w, c_proj_b,
           ln2_g, ln2_b, c_fc_w, c_fc_b, c_fc2_w, c_fc2_b,
           mlp_proj_w, mlp_proj_b, slopes):
    seq, hidden = x.shape
    num_heads = slopes.shape[0]
    head_dim = hidden // num_heads
    scale = float(head_dim) ** -1.0
    eps = 1e-5
    bf = jnp.bfloat16

    qkv = _ln_matmul_bf16(x, ln1_g, ln1_b, c_attn_w.astype(bf), c_attn_b,
                          eps=eps)
    ctx = _attention(qkv, slopes, num_heads=num_heads, head_dim=head_dim,
                     scale=scale)
    return _block2(ctx, c_proj_w.astype(bf), c_proj_b, x, ln2_g, ln2_b,
                   c_fc_w.astype(bf), c_fc_b, c_fc2_w.astype(bf), c_fc2_b,
                   mlp_proj_w.astype(bf), mlp_proj_b, eps=eps)
